# Initial kernel scaffold; baseline (speedup 1.0000x reference)
import jax
import jax.numpy as jnp
from jax.experimental import pallas as pl


def kernel(logits, x_t):
    raise NotImplementedError("write your pallas kernel here")


# speedup vs baseline: 2.2762x; 2.2762x over previous
import numpy as np
import jax
import jax.numpy as jnp
from jax.experimental import pallas as pl
from jax.experimental.pallas import tpu as pltpu

VOCAB = 100000
SEQ = 32
BATCH = 16
ROWS = BATCH * SEQ
MASK_ID = VOCAB - 1
KSEL = max(1, SEQ // 8)
RB = 8
SUBL = 8
VSUB = VOCAB // SUBL

U32 = jnp.uint32
_TINY = np.float32(np.finfo(np.float32).tiny)


def _conf_body(l_ref, out_ref):
    l = l_ref[...]
    col = jax.lax.broadcasted_iota(jnp.int32, (RB, VOCAB), 1)
    l = jnp.where(col == MASK_ID, -jnp.inf, l)
    m = jnp.max(l, axis=1, keepdims=True)
    s = jnp.sum(jnp.exp(l - m), axis=1)
    out_ref[0, 0, :] = 1.0 / s


def _select_body(conf_ref, xt_ref, pos_ref):
    conf = conf_ref[...]
    xt = xt_ref[...]
    cm = xt == MASK_ID
    mc = jnp.where(cm, conf, -jnp.inf)
    col = jax.lax.broadcasted_iota(jnp.int32, (BATCH, SEQ), 1)
    work = mc
    thresh = None
    for _ in range(KSEL):
        thresh = jnp.max(work, axis=1, keepdims=True)
        hit = work == thresh
        first = jnp.min(jnp.where(hit, col, SEQ), axis=1, keepdims=True)
        work = jnp.where(col == first, -jnp.inf, work)
    pos = cm & (mc >= thresh)
    pos_ref[...] = pos.astype(jnp.int32)


def _rotl(x, d):
    return (x << U32(d)) | (x >> U32(32 - d))


def _sample_body(ids_ref, cnt_ref, l_ref, xb_ref, out_ref):
    slot = pl.program_id(0)
    cnt = cnt_ref[0]

    @pl.when(slot < cnt)
    def _():
        row = ids_ref[slot]
        l = l_ref[0]
        col = (jax.lax.broadcasted_iota(jnp.int32, (SUBL, VSUB), 0) * VSUB
               + jax.lax.broadcasted_iota(jnp.int32, (SUBL, VSUB), 1))
        lin = (row * VOCAB + col).astype(U32)
        ks = (U32(0), U32(42), U32(0 ^ 42 ^ 0x1BD11BDA))
        x0 = jnp.zeros((SUBL, VSUB), U32) + ks[0]
        x1 = lin + ks[1]
        rots = ((13, 15, 26, 6), (17, 29, 16, 24))
        for i in range(5):
            for d in rots[i % 2]:
                x0 = x0 + x1
                x1 = _rotl(x1, d) ^ x0
            x0 = x0 + ks[(i + 1) % 3]
            x1 = x1 + ks[(i + 2) % 3] + U32(i + 1)
        bits = x0 ^ x1
        fb = (bits >> U32(9)) | U32(0x3F800000)
        f = jax.lax.bitcast_convert_type(fb, jnp.float32) - jnp.float32(1.0)
        u = jnp.maximum(_TINY, f * jnp.float32(1.0) + _TINY)
        g = -jnp.log(-jnp.log(u))
        lv = jnp.where(col == MASK_ID, -jnp.inf, l)
        pert = g + lv
        m = jnp.max(pert)
        idx = jnp.min(jnp.where(pert == m, col, VOCAB))
        out_ref[0, 0, :] = jnp.full((SUBL,), idx, jnp.int32)

    @pl.when(cnt == 0)
    def _():
        out_ref[0, 0, :] = xb_ref[0, 0, :]


def kernel(logits, x_t):
    xt = x_t.astype(jnp.int32)
    lg2 = logits.reshape(ROWS, VOCAB)

    conf3 = pl.pallas_call(
        _conf_body,
        grid=(ROWS // RB,),
        in_specs=[pl.BlockSpec((RB, VOCAB), lambda i: (i, 0))],
        out_specs=pl.BlockSpec((1, 1, RB), lambda i: (i, 0, 0)),
        out_shape=jax.ShapeDtypeStruct((ROWS // RB, 1, RB), jnp.float32),
    )(lg2)
    conf = conf3.reshape(BATCH, SEQ)

    pos = pl.pallas_call(
        _select_body,
        in_specs=[pl.BlockSpec((BATCH, SEQ), lambda: (0, 0)),
                  pl.BlockSpec((BATCH, SEQ), lambda: (0, 0))],
        out_specs=pl.BlockSpec((BATCH, SEQ), lambda: (0, 0)),
        out_shape=jax.ShapeDtypeStruct((BATCH, SEQ), jnp.int32),
    )(conf, xt)

    posf = pos.reshape(ROWS)
    count = jnp.sum(posf).astype(jnp.int32)
    order = jnp.argsort(1 - posf, stable=True).astype(jnp.int32)
    pad_id = order[jnp.maximum(count - 1, 0)]
    slots = jnp.where(jnp.arange(ROWS, dtype=jnp.int32) < count, order, pad_id)

    lg3 = lg2.reshape(ROWS, SUBL, VSUB)
    xb = jnp.broadcast_to(xt.reshape(ROWS, 1, 1), (ROWS, 1, SUBL))

    grid_spec = pltpu.PrefetchScalarGridSpec(
        num_scalar_prefetch=2,
        grid=(ROWS,),
        in_specs=[
            pl.BlockSpec((1, SUBL, VSUB), lambda i, ids, cnt: (ids[i], 0, 0)),
            pl.BlockSpec((1, 1, SUBL), lambda i, ids, cnt: (ids[i], 0, 0)),
        ],
        out_specs=pl.BlockSpec((1, 1, SUBL),
                               lambda i, ids, cnt: (ids[i], 0, 0)),
    )
    out = pl.pallas_call(
        _sample_body,
        grid_spec=grid_spec,
        out_shape=jax.ShapeDtypeStruct((ROWS, 1, SUBL), jnp.int32),
        input_output_aliases={3: 0},
    )(slots, count[None], lg3, xb)

    x_t_new = out[:, 0, 0].reshape(BATCH, SEQ)
    return x_t_new, conf
